# baseline (speedup 1.0000x reference)
import jax
import jax.numpy as jnp
from jax import lax
from jax.experimental import pallas as pl
from jax.experimental.pallas import tpu as pltpu
from jax.experimental.pallas import tpu_sc as plsc

N = 10000
E = 320000
D = 128
NUM_ATOM = 120

NC = 1
NS = 16
L = 16
NW = NC * NS

W_OUT = D
EPW = E // NW
C = 32
NCHUNK = EPW // C
N_PAD = 10240
ZR = N_PAD // NS
ZC = 128

ROWBLK = 400
NBLK = N // ROWBLK


def _tc1_body(x_ref, emb_ref, w1a_ref, w1b_ref, be1_ref, h_ref, a_ref, b_ref):
    xb = x_ref[...][:, 0]
    iota = lax.broadcasted_iota(jnp.int32, (ROWBLK, D), 1)
    oh = (xb[:, None] == iota).astype(jnp.float32)
    h = jnp.dot(oh, emb_ref[...], preferred_element_type=jnp.float32)
    h_ref[...] = h
    a_ref[...] = jnp.dot(h, w1a_ref[...], preferred_element_type=jnp.float32) \
        + be1_ref[...]
    b_ref[...] = jnp.dot(h, w1b_ref[...], preferred_element_type=jnp.float32)


def _tc1(x, emb_pad, w1a, w1b, be1):
    out_shapes = [jax.ShapeDtypeStruct((N, D), jnp.float32)] * 3
    return pl.pallas_call(
        _tc1_body,
        grid=(NBLK,),
        in_specs=[
            pl.BlockSpec((ROWBLK, 1), lambda i: (i, 0)),
            pl.BlockSpec((D, D), lambda i: (0, 0)),
            pl.BlockSpec((D, D), lambda i: (0, 0)),
            pl.BlockSpec((D, D), lambda i: (0, 0)),
            pl.BlockSpec((1, D), lambda i: (0, 0)),
        ],
        out_specs=[pl.BlockSpec((ROWBLK, D), lambda i: (i, 0))] * 3,
        out_shape=out_shapes,
    )(x, emb_pad, w1a, w1b, be1)


def _rsqrt(s):
    i = plsc.bitcast(s, jnp.int32)
    m = jnp.full((L,), 0x5F3759DF, jnp.int32) - lax.shift_right_arithmetic(
        i, jnp.full((L,), 1, jnp.int32))
    y = plsc.bitcast(m, jnp.float32)
    y = y * (1.5 - ((0.5 * s) * y) * y)
    y = y * (1.5 - ((0.5 * s) * y) * y)
    y = y * (1.5 - ((0.5 * s) * y) * y)
    return y


def _sc_body(a_hbm, b_hbm, src_hbm, dst_hbm, pos_hbm, w3_hbm, u_hbm, out_hbm,
             pos_v, w3_v, u_v, idx_s, idx_d, dist_v, tmp_v, a_v, b_v,
             s_v, acc_sh, sem):
    cid = lax.axis_index("c")
    sid = lax.axis_index("s")
    wid = sid * NC + cid

    pltpu.sync_copy(pos_hbm, pos_v)
    pltpu.sync_copy(w3_hbm, w3_v)
    pltpu.sync_copy(u_hbm, u_v)

    def sv_zero(i, carry):
        for j in range(W_OUT // L):
            s_v[i, pl.ds(j * L, L)] = jnp.zeros((L,), jnp.float32)
        return carry
    lax.fori_loop(0, C, sv_zero, 0)
    for z in range(ZR // C):
        pltpu.sync_copy(s_v, acc_sh.at[pl.ds(sid * ZR + z * C, C)])
    plsc.subcore_barrier()

    def chunk(k, carry):
        base = wid * EPW + k * C
        pltpu.sync_copy(src_hbm.at[pl.ds(base, C)], idx_s)
        pltpu.sync_copy(dst_hbm.at[pl.ds(base, C)], idx_d)
        ca = pltpu.async_copy(a_hbm.at[idx_s], a_v, sem)
        cb = pltpu.async_copy(b_hbm.at[idx_d], b_v, sem)
        ca.wait()
        cb.wait()

        def dgrp(g, c2):
            s16 = idx_s[pl.ds(g * L, L)]
            d16 = idx_d[pl.ds(g * L, L)]
            s3 = s16 * 3
            d3 = d16 * 3
            acc = jnp.zeros((L,), jnp.float32)
            for j in range(3):
                ps = plsc.load_gather(pos_v, [s3 + j])
                pd = plsc.load_gather(pos_v, [d3 + j])
                dif = ps - pd
                acc = acc + dif * dif
            dist16 = acc * _rsqrt(acc)
            tmp_v[pl.ds(0, L)] = dist16
            for t in range(L):
                rep = plsc.load_gather(tmp_v, [jnp.full((L,), t, jnp.int32)])
                dist_v[pl.ds((g * L + t) * L, L)] = rep
            return c2
        lax.fori_loop(0, C // L, dgrp, 0)

        def erow(e, c2):
            de = dist_v[pl.ds(e * L, L)]
            for c in range(D // L):
                a16 = a_v[e, pl.ds(c * L, L)]
                b16 = b_v[e, pl.ds(c * L, L)]
                w16 = w3_v[pl.ds(c * L, L)]
                pre = a16 + b16 + de * w16
                sig = 1.0 / (1.0 + jnp.exp(-pre))
                s_v[e, pl.ds(c * L, L)] = pre * sig + u_v[pl.ds(c * L, L)]
            return c2
        lax.fori_loop(0, C, erow, 0)

        pltpu.sync_copy(s_v, acc_sh.at[idx_s], add=True)
        return carry

    lax.fori_loop(0, NCHUNK, chunk, 0)
    plsc.subcore_barrier()

    for z in range(ZR // ZC):
        r0 = sid * ZR + z * ZC
        pltpu.sync_copy(acc_sh.at[pl.ds(r0, ZC)], out_hbm.at[pl.ds(r0, ZC)])


def _sc_edges(a, b, src, dst, pos, w3, u):
    mesh = plsc.VectorSubcoreMesh(core_axis_name="c", subcore_axis_name="s",
                                  num_cores=NC, num_subcores=NS)
    f = pl.kernel(
        _sc_body,
        out_type=jax.ShapeDtypeStruct((N_PAD, W_OUT), jnp.float32),
        mesh=mesh,
        compiler_params=pltpu.CompilerParams(needs_layout_passes=False),
        scratch_types=[
            pltpu.VMEM((N * 3,), jnp.float32),
            pltpu.VMEM((D,), jnp.float32),
            pltpu.VMEM((D,), jnp.float32),
            pltpu.VMEM((C,), jnp.int32),
            pltpu.VMEM((C,), jnp.int32),
            pltpu.VMEM((C * L,), jnp.float32),
            pltpu.VMEM((L,), jnp.float32),
            pltpu.VMEM((C, D), jnp.float32),
            pltpu.VMEM((C, D), jnp.float32),
            pltpu.VMEM((C, W_OUT), jnp.float32),
            pltpu.VMEM_SHARED((N_PAD, W_OUT), jnp.float32),
            pltpu.SemaphoreType.DMA,
        ],
    )
    return f(a, b, src, dst, pos, w3, u)


def _tc2_body(s2_ref, h_ref, we2_ref, whh_ref, wha_ref, bh_ref, out_ref):
    i = pl.program_id(0)
    sil = s2_ref[...]
    agg = jnp.dot(sil, we2_ref[...], preferred_element_type=jnp.float32)
    h = h_ref[...]
    u = jnp.dot(h, whh_ref[...], preferred_element_type=jnp.float32) \
        + jnp.dot(agg, wha_ref[...], preferred_element_type=jnp.float32) \
        + bh_ref[...]
    h2 = h + u * (1.0 / (1.0 + jnp.exp(-u)))

    @pl.when(i == 0)
    def _():
        out_ref[...] = jnp.zeros_like(out_ref)

    out_ref[...] += jnp.sum(h2, axis=0, keepdims=True)

    @pl.when(i == NBLK - 1)
    def _():
        out_ref[...] *= (1.0 / N)


def _tc2(s2, h, we2, wh_h, wh_a, bh):
    return pl.pallas_call(
        _tc2_body,
        grid=(NBLK,),
        in_specs=[
            pl.BlockSpec((ROWBLK, W_OUT), lambda i: (i, 0)),
            pl.BlockSpec((ROWBLK, D), lambda i: (i, 0)),
            pl.BlockSpec((D, D), lambda i: (0, 0)),
            pl.BlockSpec((D, D), lambda i: (0, 0)),
            pl.BlockSpec((D, D), lambda i: (0, 0)),
            pl.BlockSpec((1, D), lambda i: (0, 0)),
        ],
        out_specs=pl.BlockSpec((1, D), lambda i: (0, 0)),
        out_shape=jax.ShapeDtypeStruct((1, D), jnp.float32),
        compiler_params=pltpu.CompilerParams(
            dimension_semantics=("arbitrary",)),
    )(s2, h, we2, wh_h, wh_a, bh)


def kernel(x, edge, pos, emb, We1, be1, We2, be2, Wh, bh):
    emb_pad = jnp.zeros((D, D), jnp.float32).at[:NUM_ATOM].set(emb)
    w1a = We1[:D]
    w1b = We1[D:2 * D]
    w3 = We1[2 * D]
    h, a, b = _tc1(x, emb_pad, w1a, w1b, be1.reshape(1, D))
    src = edge[0]
    dst = edge[1]
    u = jnp.linalg.solve(We2.T, be2)
    s2 = _sc_edges(a, b, src, dst, pos.reshape(-1), w3, u)
    return _tc2(s2, h, We2, Wh[:D], Wh[D:], bh.reshape(1, D))


# speedup vs baseline: 1.1564x; 1.1564x over previous
import jax
import jax.numpy as jnp
from jax import lax
from jax.experimental import pallas as pl
from jax.experimental.pallas import tpu as pltpu
from jax.experimental.pallas import tpu_sc as plsc

N = 10000
E = 320000
D = 128
NUM_ATOM = 120

NC = 1
NS = 16
L = 16
NW = NC * NS

W_OUT = D
EPW = E // NW
C = 16
NCHUNK = EPW // C
N_PAD = 10240
ZR = N_PAD // NS
ZC = 128

ROWBLK = 400
NBLK = N // ROWBLK


def _tc1_body(x_ref, emb_ref, w1a_ref, w1b_ref, be1_ref, h_ref, a_ref, b_ref):
    xb = x_ref[...][:, 0]
    iota = lax.broadcasted_iota(jnp.int32, (ROWBLK, D), 1)
    oh = (xb[:, None] == iota).astype(jnp.float32)
    h = jnp.dot(oh, emb_ref[...], preferred_element_type=jnp.float32)
    h_ref[...] = h
    a_ref[...] = jnp.dot(h, w1a_ref[...], preferred_element_type=jnp.float32) \
        + be1_ref[...]
    b_ref[...] = jnp.dot(h, w1b_ref[...], preferred_element_type=jnp.float32)


def _tc1(x, emb_pad, w1a, w1b, be1):
    out_shapes = [jax.ShapeDtypeStruct((N, D), jnp.float32)] * 3
    return pl.pallas_call(
        _tc1_body,
        grid=(NBLK,),
        in_specs=[
            pl.BlockSpec((ROWBLK, 1), lambda i: (i, 0)),
            pl.BlockSpec((D, D), lambda i: (0, 0)),
            pl.BlockSpec((D, D), lambda i: (0, 0)),
            pl.BlockSpec((D, D), lambda i: (0, 0)),
            pl.BlockSpec((1, D), lambda i: (0, 0)),
        ],
        out_specs=[pl.BlockSpec((ROWBLK, D), lambda i: (i, 0))] * 3,
        out_shape=out_shapes,
    )(x, emb_pad, w1a, w1b, be1)


def _rsqrt(s):
    i = plsc.bitcast(s, jnp.int32)
    m = jnp.full((L,), 0x5F3759DF, jnp.int32) - lax.shift_right_arithmetic(
        i, jnp.full((L,), 1, jnp.int32))
    y = plsc.bitcast(m, jnp.float32)
    y = y * (1.5 - ((0.5 * s) * y) * y)
    y = y * (1.5 - ((0.5 * s) * y) * y)
    y = y * (1.5 - ((0.5 * s) * y) * y)
    return y


def _sc_body(a_hbm, b_hbm, src_hbm, dst_hbm, pos_hbm, w3_hbm, u_hbm, out_hbm,
             pos_v, w3_v, u_v, is0, id0, is1, id1, isc0, isc1, dist_v, tmp_v,
             a0, b0, a1, b1, s0, s1, acc_sh,
             si0, si1, sg0, sg1, ss0, ss1):
    cid = lax.axis_index("c")
    sid = lax.axis_index("s")
    wid = sid * NC + cid
    M = NCHUNK // 2

    pltpu.sync_copy(pos_hbm, pos_v)
    pltpu.sync_copy(w3_hbm, w3_v)
    pltpu.sync_copy(u_hbm, u_v)

    def sv_zero(i, carry):
        for j in range(W_OUT // L):
            s0[i, pl.ds(j * L, L)] = jnp.zeros((L,), jnp.float32)
        return carry
    lax.fori_loop(0, C, sv_zero, 0)
    for z in range(ZR // C):
        pltpu.sync_copy(s0, acc_sh.at[pl.ds(sid * ZR + z * C, C)])
    plsc.subcore_barrier()

    def issue_idx(c, i_s, i_d, sem):
        base = wid * EPW + c * C
        pltpu.async_copy(src_hbm.at[pl.ds(base, C)], i_s, sem)
        pltpu.async_copy(dst_hbm.at[pl.ds(base, C)], i_d, sem)

    def wait_idx(c, i_s, i_d, sem):
        base = wid * EPW + c * C
        pltpu.make_async_copy(src_hbm.at[pl.ds(base, C)], i_s, sem).wait()
        pltpu.make_async_copy(dst_hbm.at[pl.ds(base, C)], i_d, sem).wait()

    def issue_gab(a_s, b_s, i_s, i_d, sem):
        pltpu.async_copy(a_hbm.at[i_s], a_s, sem)
        pltpu.async_copy(b_hbm.at[i_d], b_s, sem)

    def wait_gab(a_s, b_s, i_s, i_d, sem):
        pltpu.make_async_copy(a_hbm.at[i_s], a_s, sem).wait()
        pltpu.make_async_copy(b_hbm.at[i_d], b_s, sem).wait()

    def wait_scat(s_s, i_c, sem):
        pltpu.make_async_copy(s_s, acc_sh.at[i_c], sem).wait()

    def compute(a_s, b_s, i_s, i_d, i_c, s_s):
        s16 = i_s[pl.ds(0, L)]
        d16 = i_d[pl.ds(0, L)]
        s3 = s16 * 3
        d3 = d16 * 3
        acc = jnp.zeros((L,), jnp.float32)
        for j in range(3):
            ps = plsc.load_gather(pos_v, [s3 + j])
            pd = plsc.load_gather(pos_v, [d3 + j])
            dif = ps - pd
            acc = acc + dif * dif
        dist16 = acc * _rsqrt(acc)
        tmp_v[pl.ds(0, L)] = dist16
        for t in range(L):
            rep = plsc.load_gather(tmp_v, [jnp.full((L,), t, jnp.int32)])
            dist_v[pl.ds(t * L, L)] = rep
        i_c[pl.ds(0, L)] = s16

        def erow(e, c2):
            de = dist_v[pl.ds(e * L, L)]
            for c in range(D // L):
                a16 = a_s[e, pl.ds(c * L, L)]
                b16 = b_s[e, pl.ds(c * L, L)]
                w16 = w3_v[pl.ds(c * L, L)]
                pre = a16 + b16 + de * w16
                sig = 1.0 / (1.0 + jnp.exp(-pre))
                s_s[e, pl.ds(c * L, L)] = pre * sig + u_v[pl.ds(c * L, L)]
            return c2
        lax.fori_loop(0, C, erow, 0)

    issue_idx(0, is0, id0, si0)
    issue_idx(1, is1, id1, si1)
    wait_idx(0, is0, id0, si0)
    issue_gab(a0, b0, is0, id0, sg0)

    def step(m, carry):
        c0 = 2 * m
        c1 = 2 * m + 1

        @pl.when(m > 0)
        def _():
            wait_scat(s0, isc0, ss0)
        wait_gab(a0, b0, is0, id0, sg0)
        wait_idx(c1, is1, id1, si1)
        issue_gab(a1, b1, is1, id1, sg1)
        compute(a0, b0, is0, id0, isc0, s0)
        pltpu.async_copy(s0, acc_sh.at[isc0], ss0, add=True)

        @pl.when(m + 1 < M)
        def _():
            issue_idx(c0 + 2, is0, id0, si0)

        @pl.when(m > 0)
        def _():
            wait_scat(s1, isc1, ss1)
        wait_gab(a1, b1, is1, id1, sg1)

        @pl.when(m + 1 < M)
        def _():
            wait_idx(c0 + 2, is0, id0, si0)
            issue_gab(a0, b0, is0, id0, sg0)
        compute(a1, b1, is1, id1, isc1, s1)
        pltpu.async_copy(s1, acc_sh.at[isc1], ss1, add=True)

        @pl.when(m + 1 < M)
        def _():
            issue_idx(c1 + 2, is1, id1, si1)
        return carry

    lax.fori_loop(0, M, step, 0)
    wait_scat(s0, isc0, ss0)
    wait_scat(s1, isc1, ss1)
    plsc.subcore_barrier()

    for z in range(ZR // ZC):
        r0 = sid * ZR + z * ZC
        pltpu.sync_copy(acc_sh.at[pl.ds(r0, ZC)], out_hbm.at[pl.ds(r0, ZC)])


def _sc_edges(a, b, src, dst, pos, w3, u):
    mesh = plsc.VectorSubcoreMesh(core_axis_name="c", subcore_axis_name="s",
                                  num_cores=NC, num_subcores=NS)
    f = pl.kernel(
        _sc_body,
        out_type=jax.ShapeDtypeStruct((N_PAD, W_OUT), jnp.float32),
        mesh=mesh,
        compiler_params=pltpu.CompilerParams(needs_layout_passes=False),
        scratch_types=[
            pltpu.VMEM((N * 3,), jnp.float32),
            pltpu.VMEM((D,), jnp.float32),
            pltpu.VMEM((D,), jnp.float32),
            pltpu.VMEM((C,), jnp.int32),
            pltpu.VMEM((C,), jnp.int32),
            pltpu.VMEM((C,), jnp.int32),
            pltpu.VMEM((C,), jnp.int32),
            pltpu.VMEM((L,), jnp.int32),
            pltpu.VMEM((L,), jnp.int32),
            pltpu.VMEM((C * L,), jnp.float32),
            pltpu.VMEM((L,), jnp.float32),
            pltpu.VMEM((C, D), jnp.float32),
            pltpu.VMEM((C, D), jnp.float32),
            pltpu.VMEM((C, D), jnp.float32),
            pltpu.VMEM((C, D), jnp.float32),
            pltpu.VMEM((C, W_OUT), jnp.float32),
            pltpu.VMEM((C, W_OUT), jnp.float32),
            pltpu.VMEM_SHARED((N_PAD, W_OUT), jnp.float32),
            pltpu.SemaphoreType.DMA,
            pltpu.SemaphoreType.DMA,
            pltpu.SemaphoreType.DMA,
            pltpu.SemaphoreType.DMA,
            pltpu.SemaphoreType.DMA,
            pltpu.SemaphoreType.DMA,
        ],
    )
    return f(a, b, src, dst, pos, w3, u)


def _tc2_body(s2_ref, h_ref, we2_ref, whh_ref, wha_ref, bh_ref, out_ref):
    i = pl.program_id(0)
    sil = s2_ref[...]
    agg = jnp.dot(sil, we2_ref[...], preferred_element_type=jnp.float32)
    h = h_ref[...]
    u = jnp.dot(h, whh_ref[...], preferred_element_type=jnp.float32) \
        + jnp.dot(agg, wha_ref[...], preferred_element_type=jnp.float32) \
        + bh_ref[...]
    h2 = h + u * (1.0 / (1.0 + jnp.exp(-u)))

    @pl.when(i == 0)
    def _():
        out_ref[...] = jnp.zeros_like(out_ref)

    out_ref[...] += jnp.sum(h2, axis=0, keepdims=True)

    @pl.when(i == NBLK - 1)
    def _():
        out_ref[...] *= (1.0 / N)


def _tc2(s2, h, we2, wh_h, wh_a, bh):
    return pl.pallas_call(
        _tc2_body,
        grid=(NBLK,),
        in_specs=[
            pl.BlockSpec((ROWBLK, W_OUT), lambda i: (i, 0)),
            pl.BlockSpec((ROWBLK, D), lambda i: (i, 0)),
            pl.BlockSpec((D, D), lambda i: (0, 0)),
            pl.BlockSpec((D, D), lambda i: (0, 0)),
            pl.BlockSpec((D, D), lambda i: (0, 0)),
            pl.BlockSpec((1, D), lambda i: (0, 0)),
        ],
        out_specs=pl.BlockSpec((1, D), lambda i: (0, 0)),
        out_shape=jax.ShapeDtypeStruct((1, D), jnp.float32),
        compiler_params=pltpu.CompilerParams(
            dimension_semantics=("arbitrary",)),
    )(s2, h, we2, wh_h, wh_a, bh)


def kernel(x, edge, pos, emb, We1, be1, We2, be2, Wh, bh):
    emb_pad = jnp.zeros((D, D), jnp.float32).at[:NUM_ATOM].set(emb)
    w1a = We1[:D]
    w1b = We1[D:2 * D]
    w3 = We1[2 * D]
    h, a, b = _tc1(x, emb_pad, w1a, w1b, be1.reshape(1, D))
    src = edge[0]
    dst = edge[1]
    u = jnp.linalg.solve(We2.T, be2)
    s2 = _sc_edges(a, b, src, dst, pos.reshape(-1), w3, u)
    return _tc2(s2, h, We2, Wh[:D], Wh[D:], bh.reshape(1, D))


# speedup vs baseline: 1.6637x; 1.4386x over previous
import jax
import jax.numpy as jnp
from jax import lax
from jax.experimental import pallas as pl
from jax.experimental.pallas import tpu as pltpu
from jax.experimental.pallas import tpu_sc as plsc

N = 10000
E = 320000
D = 128
NUM_ATOM = 120

NC = 1
NS = 16
L = 16
NW = NC * NS

W_OUT = D
EPW = E // NW
C = 32
NCHUNK = EPW // C
N_PAD = 10112
ZR = N_PAD // NS
DP = 2 * D

ROWBLK = 400
NBLK = N // ROWBLK


def _tc1_body(x_ref, emb_ref, w1a_ref, w1b_ref, be1_ref, pos_ref, h_ref,
              a_ref, b_ref):
    xb = x_ref[...][:, 0]
    iota = lax.broadcasted_iota(jnp.int32, (ROWBLK, D), 1)
    oh = (xb[:, None] == iota).astype(jnp.float32)
    h = jnp.dot(oh, emb_ref[...], preferred_element_type=jnp.float32)
    h_ref[...] = h
    pz = jnp.concatenate(
        [pos_ref[...], jnp.zeros((ROWBLK, D - 3), jnp.float32)], axis=1)
    a = jnp.dot(h, w1a_ref[...], preferred_element_type=jnp.float32) \
        + be1_ref[...]
    b = jnp.dot(h, w1b_ref[...], preferred_element_type=jnp.float32)
    a_ref[...] = jnp.concatenate([a, pz], axis=1)
    b_ref[...] = jnp.concatenate([b, pz], axis=1)


def _tc1(x, emb_pad, w1a, w1b, be1, pos):
    out_shapes = [jax.ShapeDtypeStruct((N, D), jnp.float32),
                  jax.ShapeDtypeStruct((N, DP), jnp.float32),
                  jax.ShapeDtypeStruct((N, DP), jnp.float32)]
    return pl.pallas_call(
        _tc1_body,
        grid=(NBLK,),
        in_specs=[
            pl.BlockSpec((ROWBLK, 1), lambda i: (i, 0)),
            pl.BlockSpec((D, D), lambda i: (0, 0)),
            pl.BlockSpec((D, D), lambda i: (0, 0)),
            pl.BlockSpec((D, D), lambda i: (0, 0)),
            pl.BlockSpec((1, D), lambda i: (0, 0)),
            pl.BlockSpec((ROWBLK, 3), lambda i: (i, 0)),
        ],
        out_specs=[pl.BlockSpec((ROWBLK, D), lambda i: (i, 0)),
                   pl.BlockSpec((ROWBLK, DP), lambda i: (i, 0)),
                   pl.BlockSpec((ROWBLK, DP), lambda i: (i, 0))],
        out_shape=out_shapes,
    )(x, emb_pad, w1a, w1b, be1, pos)


def _rsqrt(s):
    i = plsc.bitcast(s, jnp.int32)
    m = jnp.full((L,), 0x5F3759DF, jnp.int32) - lax.shift_right_arithmetic(
        i, jnp.full((L,), 1, jnp.int32))
    y = plsc.bitcast(m, jnp.float32)
    y = y * (1.5 - ((0.5 * s) * y) * y)
    y = y * (1.5 - ((0.5 * s) * y) * y)
    y = y * (1.5 - ((0.5 * s) * y) * y)
    return y


def _sc_body(a_hbm, b_hbm, src_hbm, dst_hbm, w3_hbm, u_hbm, out_hbm,
             w3_v, u_v, is0, id0, is1, id1, isc, dist_v, tmp_v,
             a0, b0, a1, b1, s_v, acc_sh,
             si0, si1, sg0, sg1, ssc):
    cid = lax.axis_index("c")
    sid = lax.axis_index("s")
    wid = sid * NC + cid
    M = NCHUNK // 2

    pltpu.sync_copy(w3_hbm, w3_v)
    pltpu.sync_copy(u_hbm, u_v)

    def sv_zero(i, carry):
        for j in range(W_OUT // L):
            s_v[i, pl.ds(j * L, L)] = jnp.zeros((L,), jnp.float32)
        return carry
    lax.fori_loop(0, C, sv_zero, 0)
    for z in range(ZR // C):
        pltpu.sync_copy(s_v, acc_sh.at[pl.ds(sid * ZR + z * C, C)])
    pltpu.sync_copy(s_v.at[pl.ds(0, ZR - (ZR // C) * C)],
                    acc_sh.at[pl.ds(sid * ZR + (ZR // C) * C,
                                    ZR - (ZR // C) * C)])
    plsc.subcore_barrier()

    def issue_idx(c, i_s, i_d, sem):
        base = wid * EPW + c * C
        pltpu.async_copy(src_hbm.at[pl.ds(base, C)], i_s, sem)
        pltpu.async_copy(dst_hbm.at[pl.ds(base, C)], i_d, sem)

    def wait_idx(c, i_s, i_d, sem):
        base = wid * EPW + c * C
        pltpu.make_async_copy(src_hbm.at[pl.ds(base, C)], i_s, sem).wait()
        pltpu.make_async_copy(dst_hbm.at[pl.ds(base, C)], i_d, sem).wait()

    def issue_gab(a_s, b_s, i_s, i_d, sem):
        pltpu.async_copy(a_hbm.at[i_s], a_s, sem)
        pltpu.async_copy(b_hbm.at[i_d], b_s, sem)

    def wait_gab(a_s, b_s, i_s, i_d, sem):
        pltpu.make_async_copy(a_hbm.at[i_s], a_s, sem).wait()
        pltpu.make_async_copy(b_hbm.at[i_d], b_s, sem).wait()

    def compute(a_s, b_s, i_s, i_d):
        for g in range(C // L):
            s16 = i_s[pl.ds(g * L, L)]
            isc[pl.ds(g * L, L)] = s16
            row = lax.iota(jnp.int32, L) + g * L
            acc = jnp.zeros((L,), jnp.float32)
            for j in range(3):
                col = jnp.full((L,), D + j, jnp.int32)
                ps = plsc.load_gather(a_s, [row, col])
                pd = plsc.load_gather(b_s, [row, col])
                dif = ps - pd
                acc = acc + dif * dif
            dist16 = acc * _rsqrt(acc)
            tmp_v[pl.ds(0, L)] = dist16
            for t in range(L):
                rep = plsc.load_gather(tmp_v, [jnp.full((L,), t, jnp.int32)])
                dist_v[pl.ds((g * L + t) * L, L)] = rep

        def erow(i, c2):
            e0 = 2 * i
            e1 = 2 * i + 1
            d0 = dist_v[pl.ds(e0 * L, L)]
            d1 = dist_v[pl.ds(e1 * L, L)]
            for c in range(D // L):
                w16 = w3_v[pl.ds(c * L, L)]
                u16 = u_v[pl.ds(c * L, L)]
                p0 = a_s[e0, pl.ds(c * L, L)] + b_s[e0, pl.ds(c * L, L)] \
                    + d0 * w16
                p1 = a_s[e1, pl.ds(c * L, L)] + b_s[e1, pl.ds(c * L, L)] \
                    + d1 * w16
                g0 = 1.0 / (1.0 + jnp.exp(-p0))
                g1 = 1.0 / (1.0 + jnp.exp(-p1))
                s_v[e0, pl.ds(c * L, L)] = p0 * g0 + u16
                s_v[e1, pl.ds(c * L, L)] = p1 * g1 + u16
            return c2
        lax.fori_loop(0, C // 2, erow, 0)

    def scatter_sync(i_c):
        pltpu.sync_copy(s_v, acc_sh.at[i_c], add=True)

    issue_idx(0, is0, id0, si0)
    issue_idx(1, is1, id1, si1)
    wait_idx(0, is0, id0, si0)
    issue_gab(a0, b0, is0, id0, sg0)

    def step(m, carry):
        c0 = 2 * m
        c1 = 2 * m + 1

        wait_gab(a0, b0, is0, id0, sg0)
        wait_idx(c1, is1, id1, si1)
        issue_gab(a1, b1, is1, id1, sg1)
        compute(a0, b0, is0, id0)
        issue_idx(c0 + 2, is0, id0, si0)
        scatter_sync(isc)
        wait_gab(a1, b1, is1, id1, sg1)
        wait_idx(c0 + 2, is0, id0, si0)
        issue_gab(a0, b0, is0, id0, sg0)
        compute(a1, b1, is1, id1)

        @pl.when(m + 1 < M)
        def _():
            issue_idx(c1 + 2, is1, id1, si1)
        scatter_sync(isc)
        return carry

    lax.fori_loop(0, M, step, 0)

    wait_gab(a0, b0, is0, id0, sg0)
    compute(a0, b0, is0, id0)
    scatter_sync(isc)
    plsc.subcore_barrier()

    for z in range(ZR // 8):
        r0 = sid * ZR + z * 8
        pltpu.async_copy(acc_sh.at[pl.ds(r0, 8)], out_hbm.at[pl.ds(r0, 8)], ssc)
    for z in range(ZR // 8):
        r0 = sid * ZR + z * 8
        pltpu.make_async_copy(acc_sh.at[pl.ds(r0, 8)],
                              out_hbm.at[pl.ds(r0, 8)], ssc).wait()


def _sc_edges(a, b, src, dst, w3, u):
    mesh = plsc.VectorSubcoreMesh(core_axis_name="c", subcore_axis_name="s",
                                  num_cores=NC, num_subcores=NS)
    f = pl.kernel(
        _sc_body,
        out_type=jax.ShapeDtypeStruct((N_PAD, W_OUT), jnp.float32),
        mesh=mesh,
        compiler_params=pltpu.CompilerParams(needs_layout_passes=False),
        scratch_types=[
            pltpu.VMEM((D,), jnp.float32),
            pltpu.VMEM((D,), jnp.float32),
            pltpu.VMEM((C,), jnp.int32),
            pltpu.VMEM((C,), jnp.int32),
            pltpu.VMEM((C,), jnp.int32),
            pltpu.VMEM((C,), jnp.int32),
            pltpu.VMEM((C,), jnp.int32),
            pltpu.VMEM((C * L,), jnp.float32),
            pltpu.VMEM((L,), jnp.float32),
            pltpu.VMEM((C, DP), jnp.float32),
            pltpu.VMEM((C, DP), jnp.float32),
            pltpu.VMEM((C, DP), jnp.float32),
            pltpu.VMEM((C, DP), jnp.float32),
            pltpu.VMEM((C, W_OUT), jnp.float32),
            pltpu.VMEM_SHARED((N_PAD, W_OUT), jnp.float32),
            pltpu.SemaphoreType.DMA,
            pltpu.SemaphoreType.DMA,
            pltpu.SemaphoreType.DMA,
            pltpu.SemaphoreType.DMA,
            pltpu.SemaphoreType.DMA,
        ],
    )
    return f(a, b, src, dst, w3, u)


def _tc2_body(s2_ref, h_ref, we2_ref, whh_ref, wha_ref, bh_ref, out_ref):
    i = pl.program_id(0)
    sil = s2_ref[...]
    agg = jnp.dot(sil, we2_ref[...], preferred_element_type=jnp.float32)
    h = h_ref[...]
    u = jnp.dot(h, whh_ref[...], preferred_element_type=jnp.float32) \
        + jnp.dot(agg, wha_ref[...], preferred_element_type=jnp.float32) \
        + bh_ref[...]
    h2 = h + u * (1.0 / (1.0 + jnp.exp(-u)))

    @pl.when(i == 0)
    def _():
        out_ref[...] = jnp.zeros_like(out_ref)

    out_ref[...] += jnp.sum(h2, axis=0, keepdims=True)

    @pl.when(i == NBLK - 1)
    def _():
        out_ref[...] *= (1.0 / N)


def _tc2(s2, h, we2, wh_h, wh_a, bh):
    return pl.pallas_call(
        _tc2_body,
        grid=(NBLK,),
        in_specs=[
            pl.BlockSpec((ROWBLK, W_OUT), lambda i: (i, 0)),
            pl.BlockSpec((ROWBLK, D), lambda i: (i, 0)),
            pl.BlockSpec((D, D), lambda i: (0, 0)),
            pl.BlockSpec((D, D), lambda i: (0, 0)),
            pl.BlockSpec((D, D), lambda i: (0, 0)),
            pl.BlockSpec((1, D), lambda i: (0, 0)),
        ],
        out_specs=pl.BlockSpec((1, D), lambda i: (0, 0)),
        out_shape=jax.ShapeDtypeStruct((1, D), jnp.float32),
        compiler_params=pltpu.CompilerParams(
            dimension_semantics=("arbitrary",)),
    )(s2, h, we2, wh_h, wh_a, bh)


def kernel(x, edge, pos, emb, We1, be1, We2, be2, Wh, bh):
    emb_pad = jnp.zeros((D, D), jnp.float32).at[:NUM_ATOM].set(emb)
    w1a = We1[:D]
    w1b = We1[D:2 * D]
    w3 = We1[2 * D]
    h, a, b = _tc1(x, emb_pad, w1a, w1b, be1.reshape(1, D), pos)
    src = edge[0]
    dst = edge[1]
    u = jnp.linalg.solve(We2.T, be2)
    s2 = _sc_edges(a, b, src, dst, w3, u)
    return _tc2(s2, h, We2, Wh[:D], Wh[D:], bh.reshape(1, D))


# speedup vs baseline: 2.0994x; 1.2619x over previous
import jax
import jax.numpy as jnp
from jax import lax
from jax.experimental import pallas as pl
from jax.experimental.pallas import tpu as pltpu
from jax.experimental.pallas import tpu_sc as plsc

N = 10000
E = 320000
D = 128
NUM_ATOM = 120

NC = 1
NS = 16
L = 16
NW = NC * NS

W_OUT = D
EPW = E // NW
C = 32
NCHUNK = EPW // C
N_PAD = 10112
ZR = N_PAD // NS
DP = 2 * D

ROWBLK = 400
NBLK = N // ROWBLK


def _tc1_body(x_ref, emb_ref, w1a_ref, w1b_ref, be1_ref, pos_ref, h_ref,
              a_ref, b_ref):
    xb = x_ref[...][:, 0]
    iota = lax.broadcasted_iota(jnp.int32, (ROWBLK, D), 1)
    oh = (xb[:, None] == iota).astype(jnp.float32)
    h = jnp.dot(oh, emb_ref[...], preferred_element_type=jnp.float32)
    h_ref[...] = h
    pz = jnp.concatenate(
        [pos_ref[...], jnp.zeros((ROWBLK, D - 3), jnp.float32)], axis=1)
    a = jnp.dot(h, w1a_ref[...], preferred_element_type=jnp.float32) \
        + be1_ref[...]
    b = jnp.dot(h, w1b_ref[...], preferred_element_type=jnp.float32)
    a_ref[...] = jnp.concatenate([a, pz], axis=1)
    b_ref[...] = jnp.concatenate([b, pz], axis=1)


def _tc1(x, emb_pad, w1a, w1b, be1, pos):
    out_shapes = [jax.ShapeDtypeStruct((N, D), jnp.float32),
                  jax.ShapeDtypeStruct((N, DP), jnp.float32),
                  jax.ShapeDtypeStruct((N, DP), jnp.float32)]
    return pl.pallas_call(
        _tc1_body,
        grid=(NBLK,),
        in_specs=[
            pl.BlockSpec((ROWBLK, 1), lambda i: (i, 0)),
            pl.BlockSpec((D, D), lambda i: (0, 0)),
            pl.BlockSpec((D, D), lambda i: (0, 0)),
            pl.BlockSpec((D, D), lambda i: (0, 0)),
            pl.BlockSpec((1, D), lambda i: (0, 0)),
            pl.BlockSpec((ROWBLK, 3), lambda i: (i, 0)),
        ],
        out_specs=[pl.BlockSpec((ROWBLK, D), lambda i: (i, 0)),
                   pl.BlockSpec((ROWBLK, DP), lambda i: (i, 0)),
                   pl.BlockSpec((ROWBLK, DP), lambda i: (i, 0))],
        out_shape=out_shapes,
    )(x, emb_pad, w1a, w1b, be1, pos)


def _rsqrt(s):
    i = plsc.bitcast(s, jnp.int32)
    m = jnp.full((L,), 0x5F3759DF, jnp.int32) - lax.shift_right_arithmetic(
        i, jnp.full((L,), 1, jnp.int32))
    y = plsc.bitcast(m, jnp.float32)
    y = y * (1.5 - ((0.5 * s) * y) * y)
    y = y * (1.5 - ((0.5 * s) * y) * y)
    y = y * (1.5 - ((0.5 * s) * y) * y)
    return y


def _sc_body(a_hbm, b_hbm, src_hbm, dst_hbm, w3_hbm, u_hbm, out_hbm,
             w3_v, u_v, is0, id0, is1, id1, isc, dist_v, tmp_v,
             a0, b0, a1, b1, s_v, acc_sh,
             si0, si1, sg0, sg1, ssc):
    cid = lax.axis_index("c")
    sid = lax.axis_index("s")
    wid = sid * NC + cid
    M = NCHUNK // 2

    pltpu.sync_copy(w3_hbm, w3_v)
    pltpu.sync_copy(u_hbm, u_v)

    def sv_zero(i, carry):
        for j in range(W_OUT // L):
            s_v[i, pl.ds(j * L, L)] = jnp.zeros((L,), jnp.float32)
        return carry
    lax.fori_loop(0, C, sv_zero, 0)
    for z in range(ZR // C):
        pltpu.sync_copy(s_v, acc_sh.at[pl.ds(sid * ZR + z * C, C)])
    pltpu.sync_copy(s_v.at[pl.ds(0, ZR - (ZR // C) * C)],
                    acc_sh.at[pl.ds(sid * ZR + (ZR // C) * C,
                                    ZR - (ZR // C) * C)])
    plsc.subcore_barrier()

    def issue_idx(c, i_s, i_d, sem):
        base = wid * EPW + c * C
        pltpu.async_copy(src_hbm.at[pl.ds(base, C)], i_s, sem)
        pltpu.async_copy(dst_hbm.at[pl.ds(base, C)], i_d, sem)

    def wait_idx(c, i_s, i_d, sem):
        base = wid * EPW + c * C
        pltpu.make_async_copy(src_hbm.at[pl.ds(base, C)], i_s, sem).wait()
        pltpu.make_async_copy(dst_hbm.at[pl.ds(base, C)], i_d, sem).wait()

    def issue_gab(a_s, b_s, i_s, i_d, sem):
        pltpu.async_copy(a_hbm.at[i_s], a_s, sem)
        pltpu.async_copy(b_hbm.at[i_d], b_s, sem)

    def wait_gab(a_s, b_s, i_s, i_d, sem):
        pltpu.make_async_copy(a_hbm.at[i_s], a_s, sem).wait()
        pltpu.make_async_copy(b_hbm.at[i_d], b_s, sem).wait()

    def compute(a_s, b_s, i_s, i_d):
        for g in range(C // L):
            s16 = i_s[pl.ds(g * L, L)]
            isc[pl.ds(g * L, L)] = s16
            row = lax.iota(jnp.int32, L) + g * L
            acc = jnp.zeros((L,), jnp.float32)
            for j in range(3):
                col = jnp.full((L,), D + j, jnp.int32)
                ps = plsc.load_gather(a_s, [row, col])
                pd = plsc.load_gather(b_s, [row, col])
                dif = ps - pd
                acc = acc + dif * dif
            dist16 = acc * _rsqrt(acc)
            tmp_v[pl.ds(0, L)] = dist16
            for t in range(L):
                rep = plsc.load_gather(tmp_v, [jnp.full((L,), t, jnp.int32)])
                dist_v[pl.ds((g * L + t) * L, L)] = rep

        def erow(i, c2):
            es = [4 * i, 4 * i + 1, 4 * i + 2, 4 * i + 3]
            ds = [dist_v[pl.ds(e * L, L)] for e in es]
            for c in range(D // L):
                w16 = w3_v[pl.ds(c * L, L)]
                u16 = u_v[pl.ds(c * L, L)]
                ps = [a_s[e, pl.ds(c * L, L)] + b_s[e, pl.ds(c * L, L)]
                      + de * w16 for e, de in zip(es, ds)]
                gs = [1.0 / (1.0 + jnp.exp(-p)) for p in ps]
                for e, p, g in zip(es, ps, gs):
                    s_v[e, pl.ds(c * L, L)] = p * g + u16
            return c2
        lax.fori_loop(0, C // 4, erow, 0)

    def scatter_sync(i_c):
        pltpu.sync_copy(s_v, acc_sh.at[i_c], add=True)

    issue_idx(0, is0, id0, si0)
    issue_idx(1, is1, id1, si1)
    wait_idx(0, is0, id0, si0)
    issue_gab(a0, b0, is0, id0, sg0)

    def step(m, carry):
        c0 = 2 * m
        c1 = 2 * m + 1

        wait_gab(a0, b0, is0, id0, sg0)
        wait_idx(c1, is1, id1, si1)
        issue_gab(a1, b1, is1, id1, sg1)
        compute(a0, b0, is0, id0)
        issue_idx(c0 + 2, is0, id0, si0)
        scatter_sync(isc)
        wait_gab(a1, b1, is1, id1, sg1)
        wait_idx(c0 + 2, is0, id0, si0)
        issue_gab(a0, b0, is0, id0, sg0)
        compute(a1, b1, is1, id1)

        @pl.when(m + 1 < M)
        def _():
            issue_idx(c1 + 2, is1, id1, si1)
        scatter_sync(isc)
        return carry

    lax.fori_loop(0, M, step, 0)

    wait_gab(a0, b0, is0, id0, sg0)
    compute(a0, b0, is0, id0)
    scatter_sync(isc)
    plsc.subcore_barrier()

    for z in range(ZR // 8):
        r0 = sid * ZR + z * 8
        pltpu.async_copy(acc_sh.at[pl.ds(r0, 8)], out_hbm.at[pl.ds(r0, 8)], ssc)
    for z in range(ZR // 8):
        r0 = sid * ZR + z * 8
        pltpu.make_async_copy(acc_sh.at[pl.ds(r0, 8)],
                              out_hbm.at[pl.ds(r0, 8)], ssc).wait()


def _sc_edges(a, b, src, dst, w3, u):
    mesh = plsc.VectorSubcoreMesh(core_axis_name="c", subcore_axis_name="s",
                                  num_cores=NC, num_subcores=NS)
    f = pl.kernel(
        _sc_body,
        out_type=jax.ShapeDtypeStruct((N_PAD, W_OUT), jnp.float32),
        mesh=mesh,
        compiler_params=pltpu.CompilerParams(needs_layout_passes=False),
        scratch_types=[
            pltpu.VMEM((D,), jnp.float32),
            pltpu.VMEM((D,), jnp.float32),
            pltpu.VMEM((C,), jnp.int32),
            pltpu.VMEM((C,), jnp.int32),
            pltpu.VMEM((C,), jnp.int32),
            pltpu.VMEM((C,), jnp.int32),
            pltpu.VMEM((C,), jnp.int32),
            pltpu.VMEM((C * L,), jnp.float32),
            pltpu.VMEM((L,), jnp.float32),
            pltpu.VMEM((C, DP), jnp.float32),
            pltpu.VMEM((C, DP), jnp.float32),
            pltpu.VMEM((C, DP), jnp.float32),
            pltpu.VMEM((C, DP), jnp.float32),
            pltpu.VMEM((C, W_OUT), jnp.float32),
            pltpu.VMEM_SHARED((N_PAD, W_OUT), jnp.float32),
            pltpu.SemaphoreType.DMA,
            pltpu.SemaphoreType.DMA,
            pltpu.SemaphoreType.DMA,
            pltpu.SemaphoreType.DMA,
            pltpu.SemaphoreType.DMA,
        ],
    )
    return f(a, b, src, dst, w3, u)


def _tc2_body(s2_ref, h_ref, we2_ref, whh_ref, wha_ref, bh_ref, out_ref):
    i = pl.program_id(0)
    sil = s2_ref[...]
    agg = jnp.dot(sil, we2_ref[...], preferred_element_type=jnp.float32)
    h = h_ref[...]
    u = jnp.dot(h, whh_ref[...], preferred_element_type=jnp.float32) \
        + jnp.dot(agg, wha_ref[...], preferred_element_type=jnp.float32) \
        + bh_ref[...]
    h2 = h + u * (1.0 / (1.0 + jnp.exp(-u)))

    @pl.when(i == 0)
    def _():
        out_ref[...] = jnp.zeros_like(out_ref)

    out_ref[...] += jnp.sum(h2, axis=0, keepdims=True)

    @pl.when(i == NBLK - 1)
    def _():
        out_ref[...] *= (1.0 / N)


def _tc2(s2, h, we2, wh_h, wh_a, bh):
    return pl.pallas_call(
        _tc2_body,
        grid=(NBLK,),
        in_specs=[
            pl.BlockSpec((ROWBLK, W_OUT), lambda i: (i, 0)),
            pl.BlockSpec((ROWBLK, D), lambda i: (i, 0)),
            pl.BlockSpec((D, D), lambda i: (0, 0)),
            pl.BlockSpec((D, D), lambda i: (0, 0)),
            pl.BlockSpec((D, D), lambda i: (0, 0)),
            pl.BlockSpec((1, D), lambda i: (0, 0)),
        ],
        out_specs=pl.BlockSpec((1, D), lambda i: (0, 0)),
        out_shape=jax.ShapeDtypeStruct((1, D), jnp.float32),
        compiler_params=pltpu.CompilerParams(
            dimension_semantics=("arbitrary",)),
    )(s2, h, we2, wh_h, wh_a, bh)


def kernel(x, edge, pos, emb, We1, be1, We2, be2, Wh, bh):
    emb_pad = jnp.zeros((D, D), jnp.float32).at[:NUM_ATOM].set(emb)
    w1a = We1[:D]
    w1b = We1[D:2 * D]
    w3 = We1[2 * D]
    h, a, b = _tc1(x, emb_pad, w1a, w1b, be1.reshape(1, D), pos)
    src = edge[0]
    dst = edge[1]
    u = jnp.linalg.solve(We2.T, be2)
    s2 = _sc_edges(a, b, src, dst, w3, u)
    return _tc2(s2, h, We2, Wh[:D], Wh[D:], bh.reshape(1, D))


# speedup vs baseline: 2.3128x; 1.1016x over previous
import jax
import jax.numpy as jnp
from jax import lax
from jax.experimental import pallas as pl
from jax.experimental.pallas import tpu as pltpu
from jax.experimental.pallas import tpu_sc as plsc

N = 10000
E = 320000
D = 128
NUM_ATOM = 120

NC = 1
NS = 16
L = 16
NW = NC * NS

W_OUT = D
EPW = E // NW
C = 32
NCHUNK = EPW // C
N_PAD = 10112
ZR = N_PAD // NS
DP = 2 * D

ROWBLK = 400
NBLK = N // ROWBLK


def _tc1_body(x_ref, emb_ref, w1a_ref, w1b_ref, be1_ref, pos_ref, h_ref,
              a_ref, b_ref):
    xb = x_ref[...][:, 0]
    iota = lax.broadcasted_iota(jnp.int32, (ROWBLK, D), 1)
    oh = (xb[:, None] == iota).astype(jnp.float32)
    h = jnp.dot(oh, emb_ref[...], preferred_element_type=jnp.float32)
    h_ref[...] = h
    pz = jnp.concatenate(
        [pos_ref[...], jnp.zeros((ROWBLK, D - 3), jnp.float32)], axis=1)
    a = jnp.dot(h, w1a_ref[...], preferred_element_type=jnp.float32) \
        + be1_ref[...]
    b = jnp.dot(h, w1b_ref[...], preferred_element_type=jnp.float32)
    a_ref[...] = jnp.concatenate([a, pz], axis=1)
    b_ref[...] = jnp.concatenate([b, pz], axis=1)


def _tc1(x, emb_pad, w1a, w1b, be1, pos):
    out_shapes = [jax.ShapeDtypeStruct((N, D), jnp.float32),
                  jax.ShapeDtypeStruct((N, DP), jnp.float32),
                  jax.ShapeDtypeStruct((N, DP), jnp.float32)]
    return pl.pallas_call(
        _tc1_body,
        grid=(NBLK,),
        in_specs=[
            pl.BlockSpec((ROWBLK, 1), lambda i: (i, 0)),
            pl.BlockSpec((D, D), lambda i: (0, 0)),
            pl.BlockSpec((D, D), lambda i: (0, 0)),
            pl.BlockSpec((D, D), lambda i: (0, 0)),
            pl.BlockSpec((1, D), lambda i: (0, 0)),
            pl.BlockSpec((ROWBLK, 3), lambda i: (i, 0)),
        ],
        out_specs=[pl.BlockSpec((ROWBLK, D), lambda i: (i, 0)),
                   pl.BlockSpec((ROWBLK, DP), lambda i: (i, 0)),
                   pl.BlockSpec((ROWBLK, DP), lambda i: (i, 0))],
        out_shape=out_shapes,
    )(x, emb_pad, w1a, w1b, be1, pos)


def _rsqrt(s):
    i = plsc.bitcast(s, jnp.int32)
    m = jnp.full((L,), 0x5F3759DF, jnp.int32) - lax.shift_right_arithmetic(
        i, jnp.full((L,), 1, jnp.int32))
    y = plsc.bitcast(m, jnp.float32)
    y = y * (1.5 - ((0.5 * s) * y) * y)
    y = y * (1.5 - ((0.5 * s) * y) * y)
    y = y * (1.5 - ((0.5 * s) * y) * y)
    return y


def _sc_body(a_hbm, b_hbm, src_hbm, dst_hbm, w3_hbm, u_hbm, out_hbm,
             w3_v, u_v, is0, id0, is1, id1, isc, dist_v, tmp_v,
             a0, b0, a1, b1, s_v, acc_sh,
             si0, si1, sg0, sg1, ssc):
    cid = lax.axis_index("c")
    sid = lax.axis_index("s")
    wid = sid * NC + cid
    M = NCHUNK // 2

    pltpu.sync_copy(w3_hbm, w3_v)
    pltpu.sync_copy(u_hbm, u_v)

    def sv_zero(i, carry):
        for j in range(W_OUT // L):
            s_v[i, pl.ds(j * L, L)] = jnp.zeros((L,), jnp.float32)
        return carry
    lax.fori_loop(0, C, sv_zero, 0)
    for z in range(ZR // C):
        pltpu.sync_copy(s_v, acc_sh.at[pl.ds(sid * ZR + z * C, C)])
    pltpu.sync_copy(s_v.at[pl.ds(0, ZR - (ZR // C) * C)],
                    acc_sh.at[pl.ds(sid * ZR + (ZR // C) * C,
                                    ZR - (ZR // C) * C)])
    plsc.subcore_barrier()

    def issue_idx(c, i_s, i_d, sem):
        base = wid * EPW + c * C
        pltpu.async_copy(src_hbm.at[pl.ds(base, C)], i_s, sem)
        pltpu.async_copy(dst_hbm.at[pl.ds(base, C)], i_d, sem)

    def wait_idx(c, i_s, i_d, sem):
        base = wid * EPW + c * C
        pltpu.make_async_copy(src_hbm.at[pl.ds(base, C)], i_s, sem).wait()
        pltpu.make_async_copy(dst_hbm.at[pl.ds(base, C)], i_d, sem).wait()

    def issue_gab(a_s, b_s, i_s, i_d, sem):
        pltpu.async_copy(a_hbm.at[i_s], a_s, sem)
        pltpu.async_copy(b_hbm.at[i_d], b_s, sem)

    def wait_gab(a_s, b_s, i_s, i_d, sem):
        pltpu.make_async_copy(a_hbm.at[i_s], a_s, sem).wait()
        pltpu.make_async_copy(b_hbm.at[i_d], b_s, sem).wait()

    def compute(a_s, b_s, i_s, i_d):
        for g in range(C // L):
            s16 = i_s[pl.ds(g * L, L)]
            isc[pl.ds(g * L, L)] = s16
            row = lax.iota(jnp.int32, L) + g * L
            acc = jnp.zeros((L,), jnp.float32)
            for j in range(3):
                col = jnp.full((L,), D + j, jnp.int32)
                ps = plsc.load_gather(a_s, [row, col])
                pd = plsc.load_gather(b_s, [row, col])
                dif = ps - pd
                acc = acc + dif * dif
            dist16 = acc * _rsqrt(acc)
            tmp_v[pl.ds(0, L)] = dist16
            for t in range(L):
                rep = plsc.load_gather(tmp_v, [jnp.full((L,), t, jnp.int32)])
                dist_v[pl.ds((g * L + t) * L, L)] = rep

        def erow(i, c2):
            es = [8 * i + t for t in range(8)]
            ds = [dist_v[pl.ds(e * L, L)] for e in es]
            for c in range(D // L):
                w16 = w3_v[pl.ds(c * L, L)]
                u16 = u_v[pl.ds(c * L, L)]
                ps = [a_s[e, pl.ds(c * L, L)] + b_s[e, pl.ds(c * L, L)]
                      + de * w16 for e, de in zip(es, ds)]
                gs = [1.0 / (1.0 + jnp.exp(-p)) for p in ps]
                for e, p, g in zip(es, ps, gs):
                    s_v[e, pl.ds(c * L, L)] = p * g + u16
            return c2
        lax.fori_loop(0, C // 8, erow, 0)

    def scatter_sync(i_c):
        pltpu.sync_copy(s_v, acc_sh.at[i_c], add=True)

    issue_idx(0, is0, id0, si0)
    issue_idx(1, is1, id1, si1)
    wait_idx(0, is0, id0, si0)
    issue_gab(a0, b0, is0, id0, sg0)

    def step(m, carry):
        c0 = 2 * m
        c1 = 2 * m + 1

        wait_gab(a0, b0, is0, id0, sg0)
        wait_idx(c1, is1, id1, si1)
        issue_gab(a1, b1, is1, id1, sg1)
        compute(a0, b0, is0, id0)
        issue_idx(c0 + 2, is0, id0, si0)
        scatter_sync(isc)
        wait_gab(a1, b1, is1, id1, sg1)
        wait_idx(c0 + 2, is0, id0, si0)
        issue_gab(a0, b0, is0, id0, sg0)
        compute(a1, b1, is1, id1)

        @pl.when(m + 1 < M)
        def _():
            issue_idx(c1 + 2, is1, id1, si1)
        scatter_sync(isc)
        return carry

    lax.fori_loop(0, M, step, 0)

    wait_gab(a0, b0, is0, id0, sg0)
    compute(a0, b0, is0, id0)
    scatter_sync(isc)
    plsc.subcore_barrier()

    for z in range(ZR // 8):
        r0 = sid * ZR + z * 8
        pltpu.async_copy(acc_sh.at[pl.ds(r0, 8)], out_hbm.at[pl.ds(r0, 8)], ssc)
    for z in range(ZR // 8):
        r0 = sid * ZR + z * 8
        pltpu.make_async_copy(acc_sh.at[pl.ds(r0, 8)],
                              out_hbm.at[pl.ds(r0, 8)], ssc).wait()


def _sc_edges(a, b, src, dst, w3, u):
    mesh = plsc.VectorSubcoreMesh(core_axis_name="c", subcore_axis_name="s",
                                  num_cores=NC, num_subcores=NS)
    f = pl.kernel(
        _sc_body,
        out_type=jax.ShapeDtypeStruct((N_PAD, W_OUT), jnp.float32),
        mesh=mesh,
        compiler_params=pltpu.CompilerParams(needs_layout_passes=False),
        scratch_types=[
            pltpu.VMEM((D,), jnp.float32),
            pltpu.VMEM((D,), jnp.float32),
            pltpu.VMEM((C,), jnp.int32),
            pltpu.VMEM((C,), jnp.int32),
            pltpu.VMEM((C,), jnp.int32),
            pltpu.VMEM((C,), jnp.int32),
            pltpu.VMEM((C,), jnp.int32),
            pltpu.VMEM((C * L,), jnp.float32),
            pltpu.VMEM((L,), jnp.float32),
            pltpu.VMEM((C, DP), jnp.float32),
            pltpu.VMEM((C, DP), jnp.float32),
            pltpu.VMEM((C, DP), jnp.float32),
            pltpu.VMEM((C, DP), jnp.float32),
            pltpu.VMEM((C, W_OUT), jnp.float32),
            pltpu.VMEM_SHARED((N_PAD, W_OUT), jnp.float32),
            pltpu.SemaphoreType.DMA,
            pltpu.SemaphoreType.DMA,
            pltpu.SemaphoreType.DMA,
            pltpu.SemaphoreType.DMA,
            pltpu.SemaphoreType.DMA,
        ],
    )
    return f(a, b, src, dst, w3, u)


def _tc2_body(s2_ref, h_ref, we2_ref, whh_ref, wha_ref, bh_ref, out_ref):
    i = pl.program_id(0)
    sil = s2_ref[...]
    agg = jnp.dot(sil, we2_ref[...], preferred_element_type=jnp.float32)
    h = h_ref[...]
    u = jnp.dot(h, whh_ref[...], preferred_element_type=jnp.float32) \
        + jnp.dot(agg, wha_ref[...], preferred_element_type=jnp.float32) \
        + bh_ref[...]
    h2 = h + u * (1.0 / (1.0 + jnp.exp(-u)))

    @pl.when(i == 0)
    def _():
        out_ref[...] = jnp.zeros_like(out_ref)

    out_ref[...] += jnp.sum(h2, axis=0, keepdims=True)

    @pl.when(i == NBLK - 1)
    def _():
        out_ref[...] *= (1.0 / N)


def _tc2(s2, h, we2, wh_h, wh_a, bh):
    return pl.pallas_call(
        _tc2_body,
        grid=(NBLK,),
        in_specs=[
            pl.BlockSpec((ROWBLK, W_OUT), lambda i: (i, 0)),
            pl.BlockSpec((ROWBLK, D), lambda i: (i, 0)),
            pl.BlockSpec((D, D), lambda i: (0, 0)),
            pl.BlockSpec((D, D), lambda i: (0, 0)),
            pl.BlockSpec((D, D), lambda i: (0, 0)),
            pl.BlockSpec((1, D), lambda i: (0, 0)),
        ],
        out_specs=pl.BlockSpec((1, D), lambda i: (0, 0)),
        out_shape=jax.ShapeDtypeStruct((1, D), jnp.float32),
        compiler_params=pltpu.CompilerParams(
            dimension_semantics=("arbitrary",)),
    )(s2, h, we2, wh_h, wh_a, bh)


def kernel(x, edge, pos, emb, We1, be1, We2, be2, Wh, bh):
    emb_pad = jnp.zeros((D, D), jnp.float32).at[:NUM_ATOM].set(emb)
    w1a = We1[:D]
    w1b = We1[D:2 * D]
    w3 = We1[2 * D]
    h, a, b = _tc1(x, emb_pad, w1a, w1b, be1.reshape(1, D), pos)
    src = edge[0]
    dst = edge[1]
    u = jnp.linalg.solve(We2.T, be2)
    s2 = _sc_edges(a, b, src, dst, w3, u)
    return _tc2(s2, h, We2, Wh[:D], Wh[D:], bh.reshape(1, D))
